# Initial kernel scaffold; baseline (speedup 1.0000x reference)
import jax
import jax.numpy as jnp
from jax.experimental import pallas as pl


def kernel(v):
    raise NotImplementedError("write your pallas kernel here")


# speedup vs baseline: 35.6414x; 35.6414x over previous
import functools

import jax
import jax.numpy as jnp
from jax import lax
from jax.experimental import pallas as pl
from jax.experimental.pallas import tpu as pltpu
from jax.experimental.pallas import tpu_sc as plsc

B, L, D = 4, 4096, 1024
C = 128
R = 128
RING = 640
NRB = L // R
NCB = D // C
NW = 32
NCHUNK = NRB + 3


def _sc_call(v):
    mesh = plsc.VectorSubcoreMesh(core_axis_name="c", subcore_axis_name="s")

    @functools.partial(
        pl.kernel,
        mesh=mesh,
        out_type=jax.ShapeDtypeStruct((B, L, D), jnp.float32),
        compiler_params=pltpu.CompilerParams(needs_layout_passes=False),
        scratch_types=[
            pltpu.VMEM((RING, C), jnp.float32),
            pltpu.VMEM((R, C), jnp.float32),
            pltpu.SemaphoreType.DMA,
        ],
    )
    def k(v_hbm, out_hbm, ring, obuf, sem):
        wid = lax.axis_index("s") * 2 + lax.axis_index("c")
        b = wid // NCB
        cb = wid % NCB
        lane = lax.iota(jnp.int32, 16)

        hbm0 = -256 * cb - 256

        def fire(c):
            st = lax.rem(hbm0 + 128 * c + 2 * L, L)
            return pltpu.async_copy(
                v_hbm.at[b, pl.ds(st, 128), pl.ds(cb * C, C)],
                ring.at[pl.ds(128 * lax.rem(c, 5), 128)],
                sem,
            )

        primed = [fire(c) for c in range(4)]
        for cp in primed:
            cp.wait()

        pqs = []
        cols = []
        for g in range(C // 16):
            jl = 16 * g + lane
            d = jnp.maximum(256 * cb + 2 * jl - 2, 0)
            pqs.append((256 * cb + 256) - d)
            cols.append(jl)

        def rb_body(rb, carry):
            @pl.when(rb < NRB - 1)
            def _():
                fire(rb + 4)

            def row_body(kk, _):
                r = 2 * kk
                nb = lax.rem(128 * rb + r, RING)
                for g in range(C // 16):
                    nv = pqs[g] + nb
                    nv = jnp.where(nv >= RING, nv - RING, nv)
                    a = plsc.load_gather(ring, [nv, cols[g]])
                    bb = plsc.load_gather(ring, [nv + 1, cols[g]])
                    obuf[r, pl.ds(16 * g, 16)] = jnp.minimum(a, bb)
                    obuf[r + 1, pl.ds(16 * g, 16)] = jnp.maximum(a, bb)
                return _

            lax.fori_loop(0, R // 2, row_body, None)

            pltpu.sync_copy(
                obuf, out_hbm.at[b, pl.ds(R * rb, R), pl.ds(cb * C, C)]
            )

            @pl.when(rb < NRB - 1)
            def _():
                pltpu.make_async_copy(
                    v_hbm.at[0, pl.ds(0, 128), pl.ds(0, C)],
                    ring.at[pl.ds(0, 128)],
                    sem,
                ).wait()

            return carry

        lax.fori_loop(0, NRB, rb_body, None)

    return k(v)


def kernel(v):
    return _sc_call(v)


# speedup vs baseline: 41.6257x; 1.1679x over previous
import functools

import jax
import jax.numpy as jnp
from jax import lax
from jax.experimental import pallas as pl
from jax.experimental.pallas import tpu as pltpu
from jax.experimental.pallas import tpu_sc as plsc

B, L, D = 4, 4096, 1024
C = 128
R = 128
RING = 640
NRB = L // R
NCB = D // C
NW = 32
NCHUNK = NRB + 3


def _sc_call(v):
    mesh = plsc.VectorSubcoreMesh(core_axis_name="c", subcore_axis_name="s")

    @functools.partial(
        pl.kernel,
        mesh=mesh,
        out_type=jax.ShapeDtypeStruct((B, L, D), jnp.float32),
        compiler_params=pltpu.CompilerParams(needs_layout_passes=False),
        scratch_types=[
            pltpu.VMEM((RING, C), jnp.float32),
            pltpu.VMEM((R, C), jnp.float32),
            pltpu.VMEM((R, C), jnp.float32),
            pltpu.SemaphoreType.DMA,
            pltpu.SemaphoreType.DMA,
            pltpu.SemaphoreType.DMA,
        ],
    )
    def k(v_hbm, out_hbm, ring, obuf0, obuf1, sem, osem0, osem1):
        wid = lax.axis_index("s") * 2 + lax.axis_index("c")
        b = wid // NCB
        cb = wid % NCB
        lane = lax.iota(jnp.int32, 16)

        hbm0 = -256 * cb - 256

        def fire(c):
            st = lax.rem(hbm0 + 128 * c + 2 * L, L)
            return pltpu.async_copy(
                v_hbm.at[b, pl.ds(st, 128), pl.ds(cb * C, C)],
                ring.at[pl.ds(128 * lax.rem(c, 5), 128)],
                sem,
            )

        primed = [fire(c) for c in range(4)]
        for cp in primed:
            cp.wait()

        pqs = []
        cols = []
        for g in range(C // 16):
            jl = 16 * g + lane
            d = jnp.maximum(256 * cb + 2 * jl - 2, 0)
            pqs.append((256 * cb + 256) - d)
            cols.append(jl)

        bufs = (obuf0, obuf1)
        osems = (osem0, osem1)

        def compute_block(rb, ob):
            def row_body(kk, _):
                r = 2 * kk
                nb = lax.rem(128 * rb + r, RING)
                for g in range(C // 16):
                    nv = pqs[g] + nb
                    nv = jnp.where(nv >= RING, nv - RING, nv)
                    a = plsc.load_gather(ring, [nv, cols[g]])
                    bb = plsc.load_gather(ring, [nv + 1, cols[g]])
                    ob[r, pl.ds(16 * g, 16)] = jnp.minimum(a, bb)
                    ob[r + 1, pl.ds(16 * g, 16)] = jnp.maximum(a, bb)
                return _

            lax.fori_loop(0, R // 2, row_body, None)

        def pair_body(t, carry):
            for par in range(2):
                rb = 2 * t + par
                ob, osem = bufs[par], osems[par]

                @pl.when(rb < NRB - 1)
                def _():
                    fire(rb + 4)

                @pl.when(t >= 1)
                def _():
                    pltpu.make_async_copy(
                        ob, out_hbm.at[0, pl.ds(0, R), pl.ds(0, C)], osem
                    ).wait()

                compute_block(rb, ob)

                pltpu.async_copy(
                    ob, out_hbm.at[b, pl.ds(R * rb, R), pl.ds(cb * C, C)], osem
                )

                @pl.when(rb < NRB - 1)
                def _():
                    pltpu.make_async_copy(
                        v_hbm.at[0, pl.ds(0, 128), pl.ds(0, C)],
                        ring.at[pl.ds(0, 128)],
                        sem,
                    ).wait()

            return carry

        lax.fori_loop(0, NRB // 2, pair_body, None)

        for par in range(2):
            pltpu.make_async_copy(
                bufs[par], out_hbm.at[0, pl.ds(0, R), pl.ds(0, C)], osems[par]
            ).wait()

    return k(v)


def kernel(v):
    return _sc_call(v)
